# Initial kernel scaffold; baseline (speedup 1.0000x reference)
import jax
import jax.numpy as jnp
from jax.experimental import pallas as pl


def kernel(x, edge_index, Ws, a_att, W_out, a_out):
    raise NotImplementedError("write your pallas kernel here")


# speedup vs baseline: 29.7079x; 29.7079x over previous
import functools

import jax
import jax.numpy as jnp
from jax import lax
from jax.experimental import pallas as pl
from jax.experimental.pallas import tpu as pltpu
from jax.experimental.pallas import tpu_sc as plsc

_N = 10000
_E = 320000
_F = 128
_HID = 16
_NH = 8
_NC = 40
_ALPHA = 0.2

_NCORE = 2
_NSUB = 16
_NW = _NCORE * _NSUB
_EPW = _E // _NW
_CH = 80
_NCHUNK = _EPW // _CH
_RPB = 624
_RTAIL = _N - _NSUB * _RPB

_W1 = 144
_W2 = 48


def _make_sc_edge(wt, chunk_heads):
    nck = wt // 16
    assert len(chunk_heads) == nck

    mesh = plsc.VectorSubcoreMesh(core_axis_name="c", subcore_axis_name="s")

    @functools.partial(
        pl.kernel,
        out_type=jax.ShapeDtypeStruct((_NCORE, _N, wt), jnp.float32),
        mesh=mesh,
        compiler_params=pltpu.CompilerParams(use_tc_tiling_on_sc=False),
        scratch_types=[
            pltpu.VMEM((_CH,), jnp.int32),
            pltpu.VMEM((_CH,), jnp.int32),
            pltpu.VMEM((_CH, 16), jnp.float32),
            pltpu.VMEM((_CH, 16), jnp.float32),
            pltpu.VMEM((_CH, wt), jnp.float32),
            pltpu.VMEM((_CH, wt), jnp.float32),
            pltpu.VMEM_SHARED((_N, wt), jnp.float32),
        ],
    )
    def sc_edge(src_hbm, dst_hbm, s1_hbm, s2_hbm, t_hbm, out_hbm,
                srcv, dstv, s1r, s2r, tr, comp, acc):
        c = lax.axis_index("c")
        s = lax.axis_index("s")
        w = c * _NSUB + s

        def zrow(i, carry):
            for j in range(nck):
                comp[i, pl.ds(16 * j, 16)] = jnp.zeros((16,), jnp.float32)
            return carry
        lax.fori_loop(0, _CH, zrow, 0)

        r0 = s * _RPB
        off = 0
        while off < _RPB:
            n = min(_CH, _RPB - off)
            pltpu.sync_copy(comp.at[pl.ds(0, n)], acc.at[pl.ds(r0 + off, n)])
            off += n

        @pl.when(s == _NSUB - 1)
        def _clear_tail():
            pltpu.sync_copy(comp.at[pl.ds(0, _RTAIL)],
                            acc.at[pl.ds(_NSUB * _RPB, _RTAIL)])
        plsc.subcore_barrier()

        base = w * _EPW

        def chunk(i, carry):
            eo = base + i * _CH
            pltpu.sync_copy(src_hbm.at[pl.ds(eo, _CH)], srcv)
            pltpu.sync_copy(dst_hbm.at[pl.ds(eo, _CH)], dstv)
            pltpu.sync_copy(s1_hbm.at[srcv], s1r)
            pltpu.sync_copy(s2_hbm.at[dstv], s2r)
            pltpu.sync_copy(t_hbm.at[srcv], tr)

            def edge(e, ecarry):
                ev = s1r[e] + s2r[e]
                ev = jnp.where(ev >= 0.0, ev, _ALPHA * ev)
                exv = jnp.exp(ev)
                for j, hk in enumerate(chunk_heads):
                    t = tr[e, pl.ds(16 * j, 16)]
                    if hk is None:
                        comp[e, pl.ds(16 * j, 16)] = exv * t
                    else:
                        comp[e, pl.ds(16 * j, 16)] = exv[hk] * t
                return ecarry
            lax.fori_loop(0, _CH, edge, 0)

            pltpu.sync_copy(comp, acc.at[dstv], add=True)
            return carry
        lax.fori_loop(0, _NCHUNK, chunk, 0)

        plsc.subcore_barrier()
        off = 0
        while off < _RPB:
            n = min(_CH, _RPB - off)
            pltpu.sync_copy(acc.at[pl.ds(r0 + off, n)],
                            out_hbm.at[c, pl.ds(r0 + off, n)])
            off += n

        @pl.when(s == _NSUB - 1)
        def _out_tail():
            pltpu.sync_copy(acc.at[pl.ds(_NSUB * _RPB, _RTAIL)],
                            out_hbm.at[c, pl.ds(_NSUB * _RPB, _RTAIL)])

    return sc_edge


_sc_edge_l1 = _make_sc_edge(_W1, [0, 1, 2, 3, 4, 5, 6, 7, None])
_sc_edge_l2 = _make_sc_edge(_W2, [0, 0, 0])

_BN = 2000


def _tc1_body(x_ref, wpad_ref, b_ref, m1_ref, m2_ref,
              h_ref, s1_ref, s2_ref):
    xx = x_ref[...]
    h_ref[...] = (jnp.dot(xx, wpad_ref[...],
                          preferred_element_type=jnp.float32) + b_ref[...])
    s1_ref[...] = jnp.dot(xx, m1_ref[...], preferred_element_type=jnp.float32)
    s2_ref[...] = jnp.dot(xx, m2_ref[...], preferred_element_type=jnp.float32)


def _tc1(x, wpad, b1, m1, m2):
    return pl.pallas_call(
        _tc1_body,
        grid=(_N // _BN,),
        in_specs=[
            pl.BlockSpec((_BN, _F), lambda i: (i, 0)),
            pl.BlockSpec((_F, _W1), lambda i: (0, 0)),
            pl.BlockSpec((1, _W1), lambda i: (0, 0)),
            pl.BlockSpec((_F, 16), lambda i: (0, 0)),
            pl.BlockSpec((_F, 16), lambda i: (0, 0)),
        ],
        out_specs=[
            pl.BlockSpec((_BN, _W1), lambda i: (i, 0)),
            pl.BlockSpec((_BN, 16), lambda i: (i, 0)),
            pl.BlockSpec((_BN, 16), lambda i: (i, 0)),
        ],
        out_shape=[
            jax.ShapeDtypeStruct((_N, _W1), jnp.float32),
            jax.ShapeDtypeStruct((_N, 16), jnp.float32),
            jax.ShapeDtypeStruct((_N, 16), jnp.float32),
        ],
    )(x, wpad, b1, m1, m2)


def _tc2_body(p_ref, r8_ref, w2_ref, b2_ref, m1_ref, m2_ref,
              h2_ref, s1_ref, s2_ref):
    nsum = p_ref[0] + p_ref[1]
    num = nsum[:, :_F]
    den8 = nsum[:, _F:_F + _NH]
    den = jnp.dot(den8, r8_ref[...], preferred_element_type=jnp.float32)
    feat = num / (den + 1e-16)
    feat = jnp.where(feat > 0.0, feat, jnp.exp(feat) - 1.0)
    h2_ref[...] = (jnp.dot(feat, w2_ref[...],
                           preferred_element_type=jnp.float32) + b2_ref[...])
    s1_ref[...] = jnp.dot(feat, m1_ref[...], preferred_element_type=jnp.float32)
    s2_ref[...] = jnp.dot(feat, m2_ref[...], preferred_element_type=jnp.float32)


def _tc2(p1, r8, w2pad, b2, m1b, m2b):
    return pl.pallas_call(
        _tc2_body,
        grid=(_N // _BN,),
        in_specs=[
            pl.BlockSpec((_NCORE, _BN, _W1), lambda i: (0, i, 0)),
            pl.BlockSpec((_NH, _F), lambda i: (0, 0)),
            pl.BlockSpec((_F, _W2), lambda i: (0, 0)),
            pl.BlockSpec((1, _W2), lambda i: (0, 0)),
            pl.BlockSpec((_F, 16), lambda i: (0, 0)),
            pl.BlockSpec((_F, 16), lambda i: (0, 0)),
        ],
        out_specs=[
            pl.BlockSpec((_BN, _W2), lambda i: (i, 0)),
            pl.BlockSpec((_BN, 16), lambda i: (i, 0)),
            pl.BlockSpec((_BN, 16), lambda i: (i, 0)),
        ],
        out_shape=[
            jax.ShapeDtypeStruct((_N, _W2), jnp.float32),
            jax.ShapeDtypeStruct((_N, 16), jnp.float32),
            jax.ShapeDtypeStruct((_N, 16), jnp.float32),
        ],
    )(p1, r8, w2pad, b2, m1b, m2b)


def _tc3_body(p_ref, o_ref):
    n2 = p_ref[0] + p_ref[1]
    num = n2[:, :_NC]
    den = n2[:, _NC:_NC + 1]
    o = num / (den + 1e-16)
    o = jnp.where(o > 0.0, o, jnp.exp(o) - 1.0)
    pooled = jnp.mean(o, axis=0, keepdims=True)
    m = jnp.max(pooled, axis=1, keepdims=True)
    z = pooled - m
    lse = jnp.log(jnp.sum(jnp.exp(z), axis=1, keepdims=True))
    o_ref[...] = z - lse


def _tc3(p2):
    return pl.pallas_call(
        _tc3_body,
        out_shape=jax.ShapeDtypeStruct((1, _NC), jnp.float32),
    )(p2)


def kernel(x, edge_index, Ws, a_att, W_out, a_out):
    f32 = jnp.float32
    src = edge_index[0].astype(jnp.int32)
    dst = edge_index[1].astype(jnp.int32)

    wcat = jnp.transpose(Ws, (1, 0, 2)).reshape(_F, _NH * _HID)
    wpad = jnp.pad(wcat, ((0, 0), (0, _W1 - _NH * _HID)))
    b1 = jnp.zeros((1, _W1), f32).at[0, _F:_F + _NH].set(1.0)
    a1 = a_att[:, :_HID, 0]
    a2 = a_att[:, _HID:, 0]
    m1 = jnp.pad(jnp.einsum("kfj,kj->fk", Ws, a1), ((0, 0), (0, 16 - _NH)))
    m2 = jnp.pad(jnp.einsum("kfj,kj->fk", Ws, a2), ((0, 0), (0, 16 - _NH)))
    r8 = jnp.repeat(jnp.eye(_NH, dtype=f32), _HID, axis=1)

    w2pad = jnp.pad(W_out, ((0, 0), (0, _W2 - _NC)))
    b2 = jnp.zeros((1, _W2), f32).at[0, _NC].set(1.0)
    m1b = jnp.pad((W_out @ a_out[:_NC, 0])[:, None], ((0, 0), (0, 15)))
    m2b = jnp.pad((W_out @ a_out[_NC:, 0])[:, None], ((0, 0), (0, 15)))

    h1t, s1, s2 = _tc1(x, wpad, b1, m1, m2)
    p1 = _sc_edge_l1(src, dst, s1, s2, h1t)
    h2t, s1b, s2b = _tc2(p1, r8, w2pad, b2, m1b, m2b)
    p2 = _sc_edge_l2(src, dst, s1b, s2b, h2t)
    return _tc3(p2)


# speedup vs baseline: 51.4556x; 1.7320x over previous
import functools

import jax
import jax.numpy as jnp
from jax import lax
from jax.experimental import pallas as pl
from jax.experimental.pallas import tpu as pltpu
from jax.experimental.pallas import tpu_sc as plsc

_N = 10000
_E = 320000
_F = 128
_HID = 16
_NH = 8
_NC = 40
_ALPHA = 0.2

_NCORE = 2
_NSUB = 16
_NW = _NCORE * _NSUB
_EPW = _E // _NW
_CH = 40
_NCHUNK = _EPW // _CH
_RPB = 624
_RTAIL = _N - _NSUB * _RPB

_A1 = 144
_T1 = 160
_A2 = 48
_T2 = 64


def _make_sc_edge(wtab, wacc, chunk_heads):
    nck = wacc // 16
    assert len(chunk_heads) == nck
    assert _NCHUNK % 2 == 0

    mesh = plsc.VectorSubcoreMesh(core_axis_name="c", subcore_axis_name="s")

    @functools.partial(
        pl.kernel,
        out_type=jax.ShapeDtypeStruct((_NCORE, _N, wacc), jnp.float32),
        mesh=mesh,
        compiler_params=pltpu.CompilerParams(use_tc_tiling_on_sc=False),
        scratch_types=[
            pltpu.VMEM((_NCHUNK, _CH), jnp.int32),
            pltpu.VMEM((_NCHUNK, _CH), jnp.int32),
            pltpu.VMEM((_CH, 16), jnp.float32),
            pltpu.VMEM((_CH, wtab), jnp.float32),
            pltpu.VMEM((_CH, 16), jnp.float32),
            pltpu.VMEM((_CH, wtab), jnp.float32),
            pltpu.VMEM((_CH, wacc), jnp.float32),
            pltpu.SemaphoreType.DMA,
            pltpu.SemaphoreType.DMA,
            pltpu.SemaphoreType.DMA,
            pltpu.VMEM_SHARED((_N, wacc), jnp.float32),
        ],
    )
    def sc_edge(src_hbm, dst_hbm, s2_hbm, t_hbm, out_hbm,
                srcv, dstv, s2r0, tr0, s2r1, tr1, comp,
                gsem0, gsem1, ssem, acc):
        c = lax.axis_index("c")
        s = lax.axis_index("s")
        w = c * _NSUB + s

        bufs = ((s2r0, tr0, gsem0), (s2r1, tr1, gsem1))

        def start_gather(ci, b):
            s2r, tr, gsem = b
            pltpu.async_copy(s2_hbm.at[dstv.at[ci]], s2r, gsem)
            pltpu.async_copy(t_hbm.at[srcv.at[ci]], tr, gsem)

        def wait_gather(ci, b):
            s2r, tr, gsem = b
            pltpu.make_async_copy(s2_hbm.at[dstv.at[ci]], s2r, gsem).wait()
            pltpu.make_async_copy(t_hbm.at[srcv.at[ci]], tr, gsem).wait()

        def start_scatter(ci):
            pltpu.async_copy(comp, acc.at[dstv.at[ci]], ssem, add=True)

        def wait_scatter(ci):
            pltpu.make_async_copy(comp, acc.at[dstv.at[ci]], ssem).wait()

        def compute(b):
            s2r, tr, _ = b

            def edge(e, ecarry):
                ev = tr[e, pl.ds(wacc, 16)] + s2r[e]
                ev = jnp.where(ev >= 0.0, ev, _ALPHA * ev)
                exv = jnp.exp(ev)
                for j, hk in enumerate(chunk_heads):
                    t = tr[e, pl.ds(16 * j, 16)]
                    if hk is None:
                        comp[e, pl.ds(16 * j, 16)] = exv * t
                    else:
                        comp[e, pl.ds(16 * j, 16)] = exv[hk] * t
                return ecarry
            lax.fori_loop(0, _CH, edge, 0)

        def zrow(i, carry):
            for j in range(nck):
                comp[i, pl.ds(16 * j, 16)] = jnp.zeros((16,), jnp.float32)
            return carry
        lax.fori_loop(0, _CH, zrow, 0)

        r0 = s * _RPB
        off = 0
        while off < _RPB:
            n = min(_CH, _RPB - off)
            pltpu.sync_copy(comp.at[pl.ds(0, n)], acc.at[pl.ds(r0 + off, n)])
            off += n

        @pl.when(s == _NSUB - 1)
        def _clear_tail():
            pltpu.sync_copy(comp.at[pl.ds(0, _RTAIL)],
                            acc.at[pl.ds(_NSUB * _RPB, _RTAIL)])

        pltpu.sync_copy(src_hbm.at[pl.ds(w * _NCHUNK, _NCHUNK)], srcv)
        pltpu.sync_copy(dst_hbm.at[pl.ds(w * _NCHUNK, _NCHUNK)], dstv)
        plsc.subcore_barrier()

        start_scatter(0)
        start_gather(0, bufs[0])
        start_gather(1, bufs[1])

        def pipe(i, carry):
            c0 = i * 2
            c1 = c0 + 1
            wait_gather(c0, bufs[0])
            wait_scatter(c0)
            compute(bufs[0])
            start_scatter(c0)

            @pl.when(i < (_NCHUNK // 2) - 1)
            def _pf0():
                start_gather(c0 + 2, bufs[0])

            wait_gather(c1, bufs[1])
            wait_scatter(c1)
            compute(bufs[1])
            start_scatter(c1)

            @pl.when(i < (_NCHUNK // 2) - 1)
            def _pf1():
                start_gather(c1 + 2, bufs[1])
            return carry
        lax.fori_loop(0, _NCHUNK // 2, pipe, 0)

        wait_scatter(_NCHUNK - 1)
        plsc.subcore_barrier()
        off = 0
        while off < _RPB:
            n = min(_CH, _RPB - off)
            pltpu.sync_copy(acc.at[pl.ds(r0 + off, n)],
                            out_hbm.at[c, pl.ds(r0 + off, n)])
            off += n

        @pl.when(s == _NSUB - 1)
        def _out_tail():
            pltpu.sync_copy(acc.at[pl.ds(_NSUB * _RPB, _RTAIL)],
                            out_hbm.at[c, pl.ds(_NSUB * _RPB, _RTAIL)])

    return sc_edge


_sc_edge_l1 = _make_sc_edge(_T1, _A1, [0, 1, 2, 3, 4, 5, 6, 7, None])
_sc_edge_l2 = _make_sc_edge(_T2, _A2, [0, 0, 0])

_BN = 2000


def _tc1_body(x_ref, wpad_ref, b_ref, m2_ref, h_ref, s2_ref):
    xx = x_ref[...]
    h_ref[...] = (jnp.dot(xx, wpad_ref[...],
                          preferred_element_type=jnp.float32) + b_ref[...])
    s2_ref[...] = jnp.dot(xx, m2_ref[...], preferred_element_type=jnp.float32)


def _tc1(x, wpad, b1, m2):
    return pl.pallas_call(
        _tc1_body,
        grid=(_N // _BN,),
        in_specs=[
            pl.BlockSpec((_BN, _F), lambda i: (i, 0)),
            pl.BlockSpec((_F, _T1), lambda i: (0, 0)),
            pl.BlockSpec((1, _T1), lambda i: (0, 0)),
            pl.BlockSpec((_F, 16), lambda i: (0, 0)),
        ],
        out_specs=[
            pl.BlockSpec((_BN, _T1), lambda i: (i, 0)),
            pl.BlockSpec((_BN, 16), lambda i: (i, 0)),
        ],
        out_shape=[
            jax.ShapeDtypeStruct((_N, _T1), jnp.float32),
            jax.ShapeDtypeStruct((_N, 16), jnp.float32),
        ],
    )(x, wpad, b1, m2)


def _tc2_body(p_ref, r8_ref, w2_ref, b2_ref, m2_ref, h2_ref, s2_ref):
    nsum = p_ref[0] + p_ref[1]
    num = nsum[:, :_F]
    den8 = nsum[:, _F:_F + _NH]
    den = jnp.dot(den8, r8_ref[...], preferred_element_type=jnp.float32)
    feat = num / (den + 1e-16)
    feat = jnp.where(feat > 0.0, feat, jnp.exp(feat) - 1.0)
    h2_ref[...] = (jnp.dot(feat, w2_ref[...],
                           preferred_element_type=jnp.float32) + b2_ref[...])
    s2_ref[...] = jnp.dot(feat, m2_ref[...], preferred_element_type=jnp.float32)


def _tc2(p1, r8, w2pad, b2, m2b):
    return pl.pallas_call(
        _tc2_body,
        grid=(_N // _BN,),
        in_specs=[
            pl.BlockSpec((_NCORE, _BN, _A1), lambda i: (0, i, 0)),
            pl.BlockSpec((_NH, _F), lambda i: (0, 0)),
            pl.BlockSpec((_F, _T2), lambda i: (0, 0)),
            pl.BlockSpec((1, _T2), lambda i: (0, 0)),
            pl.BlockSpec((_F, 16), lambda i: (0, 0)),
        ],
        out_specs=[
            pl.BlockSpec((_BN, _T2), lambda i: (i, 0)),
            pl.BlockSpec((_BN, 16), lambda i: (i, 0)),
        ],
        out_shape=[
            jax.ShapeDtypeStruct((_N, _T2), jnp.float32),
            jax.ShapeDtypeStruct((_N, 16), jnp.float32),
        ],
    )(p1, r8, w2pad, b2, m2b)


def _tc3_body(p_ref, o_ref):
    n2 = p_ref[0] + p_ref[1]
    num = n2[:, :_NC]
    den = n2[:, _NC:_NC + 1]
    o = num / (den + 1e-16)
    o = jnp.where(o > 0.0, o, jnp.exp(o) - 1.0)
    pooled = jnp.mean(o, axis=0, keepdims=True)
    m = jnp.max(pooled, axis=1, keepdims=True)
    z = pooled - m
    lse = jnp.log(jnp.sum(jnp.exp(z), axis=1, keepdims=True))
    o_ref[...] = z - lse


def _tc3(p2):
    return pl.pallas_call(
        _tc3_body,
        out_shape=jax.ShapeDtypeStruct((1, _NC), jnp.float32),
    )(p2)


def kernel(x, edge_index, Ws, a_att, W_out, a_out):
    f32 = jnp.float32
    src = edge_index[0].astype(jnp.int32).reshape(_E // _CH, _CH)
    dst = edge_index[1].astype(jnp.int32).reshape(_E // _CH, _CH)

    wcat = jnp.transpose(Ws, (1, 0, 2)).reshape(_F, _NH * _HID)
    a1 = a_att[:, :_HID, 0]
    a2 = a_att[:, _HID:, 0]
    m1 = jnp.einsum("kfj,kj->fk", Ws, a1)
    m2 = jnp.pad(jnp.einsum("kfj,kj->fk", Ws, a2), ((0, 0), (0, 16 - _NH)))
    wpad = jnp.zeros((_F, _T1), f32)
    wpad = wpad.at[:, :_NH * _HID].set(wcat)
    wpad = wpad.at[:, _A1:_A1 + _NH].set(m1)
    b1 = jnp.zeros((1, _T1), f32).at[0, _NH * _HID:_NH * _HID + _NH].set(1.0)
    r8 = jnp.repeat(jnp.eye(_NH, dtype=f32), _HID, axis=1)

    w2pad = jnp.zeros((_F, _T2), f32)
    w2pad = w2pad.at[:, :_NC].set(W_out)
    w2pad = w2pad.at[:, _A2].set(W_out @ a_out[:_NC, 0])
    b2 = jnp.zeros((1, _T2), f32).at[0, _NC].set(1.0)
    m2b = jnp.pad((W_out @ a_out[_NC:, 0])[:, None], ((0, 0), (0, 15)))

    h1t, s2v = _tc1(x, wpad, b1, m2)
    p1 = _sc_edge_l1(src, dst, s2v, h1t)
    h2t, s2b = _tc2(p1, r8, w2pad, b2, m2b)
    p2 = _sc_edge_l2(src, dst, s2b, h2t)
    return _tc3(p2)


# speedup vs baseline: 122.7053x; 2.3847x over previous
import functools

import jax
import jax.numpy as jnp
from jax import lax
from jax.experimental import pallas as pl
from jax.experimental.pallas import tpu as pltpu
from jax.experimental.pallas import tpu_sc as plsc

_N = 10000
_E = 320000
_F = 128
_HID = 16
_NH = 8
_NC = 40
_ALPHA = 0.2

_NCORE = 2
_NSUB = 16
_NW = _NCORE * _NSUB
_EPW = _E // _NW
_CH = 40
_NCHUNK = _EPW // _CH
_RPB = 624
_RTAIL = _N - _NSUB * _RPB

_A1 = 144
_T1 = 160
_A2 = 48
_T2 = 64


def _make_sc_edge(wtab, wacc, chunk_heads):
    nck = wacc // 16
    assert len(chunk_heads) == nck
    assert _NCHUNK % 2 == 0

    mesh = plsc.VectorSubcoreMesh(core_axis_name="c", subcore_axis_name="s")

    @functools.partial(
        pl.kernel,
        out_type=jax.ShapeDtypeStruct((_NCORE, _N, wacc), jnp.float32),
        mesh=mesh,
        compiler_params=pltpu.CompilerParams(use_tc_tiling_on_sc=False),
        scratch_types=[
            pltpu.VMEM((_NCHUNK, _CH), jnp.int32),
            pltpu.VMEM((_NCHUNK, _CH), jnp.int32),
            pltpu.VMEM((_CH, 16), jnp.float32),
            pltpu.VMEM((_CH, wtab), jnp.float32),
            pltpu.VMEM((_CH, 16), jnp.float32),
            pltpu.VMEM((_CH, wtab), jnp.float32),
            pltpu.VMEM((_CH, wacc), jnp.float32),
            pltpu.SemaphoreType.DMA,
            pltpu.SemaphoreType.DMA,
            pltpu.SemaphoreType.DMA,
            pltpu.VMEM_SHARED((_N, wacc), jnp.float32),
        ],
    )
    def sc_edge(src_hbm, dst_hbm, s2_hbm, t_hbm, out_hbm,
                srcv, dstv, s2r0, tr0, s2r1, tr1, comp,
                gsem0, gsem1, ssem, acc):
        c = lax.axis_index("c")
        s = lax.axis_index("s")
        w = c * _NSUB + s

        bufs = ((s2r0, tr0, gsem0), (s2r1, tr1, gsem1))

        def start_gather(ci, b):
            s2r, tr, gsem = b
            pltpu.async_copy(s2_hbm.at[dstv.at[ci]], s2r, gsem)
            pltpu.async_copy(t_hbm.at[srcv.at[ci]], tr, gsem)

        def wait_gather(ci, b):
            s2r, tr, gsem = b
            pltpu.make_async_copy(s2_hbm.at[dstv.at[ci]], s2r, gsem).wait()
            pltpu.make_async_copy(t_hbm.at[srcv.at[ci]], tr, gsem).wait()

        def start_scatter(ci):
            pltpu.async_copy(comp, acc.at[dstv.at[ci]], ssem, add=True)

        def wait_scatter(ci):
            pltpu.make_async_copy(comp, acc.at[dstv.at[ci]], ssem).wait()

        def compute(b):
            s2r, tr, _ = b

            @plsc.parallel_loop(0, _CH, 1, unroll=5)
            def edge(e):
                ev = tr[e, pl.ds(wacc, 16)] + s2r[e]
                ev = jnp.maximum(ev, _ALPHA * ev)
                exv = jnp.exp(ev)
                for j, hk in enumerate(chunk_heads):
                    t = tr[e, pl.ds(16 * j, 16)]
                    if hk is None:
                        comp[e, pl.ds(16 * j, 16)] = exv * t
                    else:
                        comp[e, pl.ds(16 * j, 16)] = exv[hk] * t

        def zrow(i, carry):
            for j in range(nck):
                comp[i, pl.ds(16 * j, 16)] = jnp.zeros((16,), jnp.float32)
            return carry
        lax.fori_loop(0, _CH, zrow, 0)

        r0 = s * _RPB
        off = 0
        while off < _RPB:
            n = min(_CH, _RPB - off)
            pltpu.sync_copy(comp.at[pl.ds(0, n)], acc.at[pl.ds(r0 + off, n)])
            off += n

        @pl.when(s == _NSUB - 1)
        def _clear_tail():
            pltpu.sync_copy(comp.at[pl.ds(0, _RTAIL)],
                            acc.at[pl.ds(_NSUB * _RPB, _RTAIL)])

        pltpu.sync_copy(src_hbm.at[pl.ds(w * _NCHUNK, _NCHUNK)], srcv)
        pltpu.sync_copy(dst_hbm.at[pl.ds(w * _NCHUNK, _NCHUNK)], dstv)
        plsc.subcore_barrier()

        start_scatter(0)
        start_gather(0, bufs[0])
        start_gather(1, bufs[1])

        def pipe(i, carry):
            c0 = i * 2
            c1 = c0 + 1
            wait_gather(c0, bufs[0])
            wait_scatter(c0)
            compute(bufs[0])
            start_scatter(c0)

            @pl.when(i < (_NCHUNK // 2) - 1)
            def _pf0():
                start_gather(c0 + 2, bufs[0])

            wait_gather(c1, bufs[1])
            wait_scatter(c1)
            compute(bufs[1])
            start_scatter(c1)

            @pl.when(i < (_NCHUNK // 2) - 1)
            def _pf1():
                start_gather(c1 + 2, bufs[1])
            return carry
        lax.fori_loop(0, _NCHUNK // 2, pipe, 0)

        wait_scatter(_NCHUNK - 1)
        plsc.subcore_barrier()
        off = 0
        while off < _RPB:
            n = min(_CH, _RPB - off)
            pltpu.sync_copy(acc.at[pl.ds(r0 + off, n)],
                            out_hbm.at[c, pl.ds(r0 + off, n)])
            off += n

        @pl.when(s == _NSUB - 1)
        def _out_tail():
            pltpu.sync_copy(acc.at[pl.ds(_NSUB * _RPB, _RTAIL)],
                            out_hbm.at[c, pl.ds(_NSUB * _RPB, _RTAIL)])

    return sc_edge


_sc_edge_l1 = _make_sc_edge(_T1, _A1, [0, 1, 2, 3, 4, 5, 6, 7, None])
_sc_edge_l2 = _make_sc_edge(_T2, _A2, [0, 0, 0])

_BN = 2000


def _tc1_body(x_ref, wpad_ref, b_ref, m2_ref, h_ref, s2_ref):
    xx = x_ref[...]
    h_ref[...] = (jnp.dot(xx, wpad_ref[...],
                          preferred_element_type=jnp.float32) + b_ref[...])
    s2_ref[...] = jnp.dot(xx, m2_ref[...], preferred_element_type=jnp.float32)


def _tc1(x, wpad, b1, m2):
    return pl.pallas_call(
        _tc1_body,
        grid=(_N // _BN,),
        in_specs=[
            pl.BlockSpec((_BN, _F), lambda i: (i, 0)),
            pl.BlockSpec((_F, _T1), lambda i: (0, 0)),
            pl.BlockSpec((1, _T1), lambda i: (0, 0)),
            pl.BlockSpec((_F, 16), lambda i: (0, 0)),
        ],
        out_specs=[
            pl.BlockSpec((_BN, _T1), lambda i: (i, 0)),
            pl.BlockSpec((_BN, 16), lambda i: (i, 0)),
        ],
        out_shape=[
            jax.ShapeDtypeStruct((_N, _T1), jnp.float32),
            jax.ShapeDtypeStruct((_N, 16), jnp.float32),
        ],
    )(x, wpad, b1, m2)


def _tc2_body(p_ref, r8_ref, w2_ref, b2_ref, m2_ref, h2_ref, s2_ref):
    nsum = p_ref[0] + p_ref[1]
    num = nsum[:, :_F]
    den8 = nsum[:, _F:_F + _NH]
    den = jnp.dot(den8, r8_ref[...], preferred_element_type=jnp.float32)
    feat = num / (den + 1e-16)
    feat = jnp.where(feat > 0.0, feat, jnp.exp(feat) - 1.0)
    h2_ref[...] = (jnp.dot(feat, w2_ref[...],
                           preferred_element_type=jnp.float32) + b2_ref[...])
    s2_ref[...] = jnp.dot(feat, m2_ref[...], preferred_element_type=jnp.float32)


def _tc2(p1, r8, w2pad, b2, m2b):
    return pl.pallas_call(
        _tc2_body,
        grid=(_N // _BN,),
        in_specs=[
            pl.BlockSpec((_NCORE, _BN, _A1), lambda i: (0, i, 0)),
            pl.BlockSpec((_NH, _F), lambda i: (0, 0)),
            pl.BlockSpec((_F, _T2), lambda i: (0, 0)),
            pl.BlockSpec((1, _T2), lambda i: (0, 0)),
            pl.BlockSpec((_F, 16), lambda i: (0, 0)),
        ],
        out_specs=[
            pl.BlockSpec((_BN, _T2), lambda i: (i, 0)),
            pl.BlockSpec((_BN, 16), lambda i: (i, 0)),
        ],
        out_shape=[
            jax.ShapeDtypeStruct((_N, _T2), jnp.float32),
            jax.ShapeDtypeStruct((_N, 16), jnp.float32),
        ],
    )(p1, r8, w2pad, b2, m2b)


def _tc3_body(p_ref, o_ref):
    n2 = p_ref[0] + p_ref[1]
    num = n2[:, :_NC]
    den = n2[:, _NC:_NC + 1]
    o = num / (den + 1e-16)
    o = jnp.where(o > 0.0, o, jnp.exp(o) - 1.0)
    pooled = jnp.mean(o, axis=0, keepdims=True)
    m = jnp.max(pooled, axis=1, keepdims=True)
    z = pooled - m
    lse = jnp.log(jnp.sum(jnp.exp(z), axis=1, keepdims=True))
    o_ref[...] = z - lse


def _tc3(p2):
    return pl.pallas_call(
        _tc3_body,
        out_shape=jax.ShapeDtypeStruct((1, _NC), jnp.float32),
    )(p2)


def kernel(x, edge_index, Ws, a_att, W_out, a_out):
    f32 = jnp.float32
    src = edge_index[0].astype(jnp.int32).reshape(_E // _CH, _CH)
    dst = edge_index[1].astype(jnp.int32).reshape(_E // _CH, _CH)

    wcat = jnp.transpose(Ws, (1, 0, 2)).reshape(_F, _NH * _HID)
    a1 = a_att[:, :_HID, 0]
    a2 = a_att[:, _HID:, 0]
    m1 = jnp.einsum("kfj,kj->fk", Ws, a1)
    m2 = jnp.pad(jnp.einsum("kfj,kj->fk", Ws, a2), ((0, 0), (0, 16 - _NH)))
    wpad = jnp.zeros((_F, _T1), f32)
    wpad = wpad.at[:, :_NH * _HID].set(wcat)
    wpad = wpad.at[:, _A1:_A1 + _NH].set(m1)
    b1 = jnp.zeros((1, _T1), f32).at[0, _NH * _HID:_NH * _HID + _NH].set(1.0)
    r8 = jnp.repeat(jnp.eye(_NH, dtype=f32), _HID, axis=1)

    w2pad = jnp.zeros((_F, _T2), f32)
    w2pad = w2pad.at[:, :_NC].set(W_out)
    w2pad = w2pad.at[:, _A2].set(W_out @ a_out[:_NC, 0])
    b2 = jnp.zeros((1, _T2), f32).at[0, _NC].set(1.0)
    m2b = jnp.pad((W_out @ a_out[_NC:, 0])[:, None], ((0, 0), (0, 15)))

    h1t, s2v = _tc1(x, wpad, b1, m2)
    p1 = _sc_edge_l1(src, dst, s2v, h1t)
    h2t, s2b = _tc2(p1, r8, w2pad, b2, m2b)
    p2 = _sc_edge_l2(src, dst, s2b, h2t)
    return _tc3(p2)


# speedup vs baseline: 124.9823x; 1.0186x over previous
import functools

import jax
import jax.numpy as jnp
from jax import lax
from jax.experimental import pallas as pl
from jax.experimental.pallas import tpu as pltpu
from jax.experimental.pallas import tpu_sc as plsc

_N = 10000
_E = 320000
_F = 128
_HID = 16
_NH = 8
_NC = 40
_ALPHA = 0.2

_NCORE = 2
_NSUB = 16
_NW = _NCORE * _NSUB
_EPW = _E // _NW
_CH = 40
_NCHUNK = _EPW // _CH
_RPB = 624
_RTAIL = _N - _NSUB * _RPB

_A1 = 144
_T1 = 160
_A2 = 48
_T2 = 64


def _make_sc_edge(wtab, wacc, chunk_heads):
    nck = wacc // 16
    assert len(chunk_heads) == nck
    assert _NCHUNK % 2 == 0

    mesh = plsc.VectorSubcoreMesh(core_axis_name="c", subcore_axis_name="s")

    @functools.partial(
        pl.kernel,
        out_type=jax.ShapeDtypeStruct((_NCORE, _N, wacc), jnp.float32),
        mesh=mesh,
        compiler_params=pltpu.CompilerParams(use_tc_tiling_on_sc=False),
        scratch_types=[
            pltpu.VMEM((_NCHUNK, _CH), jnp.int32),
            pltpu.VMEM((_NCHUNK, _CH), jnp.int32),
            pltpu.VMEM((_CH, 16), jnp.float32),
            pltpu.VMEM((_CH, wtab), jnp.float32),
            pltpu.VMEM((_CH, 16), jnp.float32),
            pltpu.VMEM((_CH, wtab), jnp.float32),
            pltpu.VMEM((_CH, wacc), jnp.float32),
            pltpu.SemaphoreType.DMA,
            pltpu.SemaphoreType.DMA,
            pltpu.SemaphoreType.DMA,
            pltpu.VMEM_SHARED((_N, wacc), jnp.float32),
        ],
    )
    def sc_edge(src_hbm, dst_hbm, s2_hbm, t_hbm, out_hbm,
                srcv, dstv, s2r0, tr0, s2r1, tr1, comp,
                gsem0, gsem1, ssem, acc):
        c = lax.axis_index("c")
        s = lax.axis_index("s")
        w = c * _NSUB + s

        bufs = ((s2r0, tr0, gsem0), (s2r1, tr1, gsem1))

        def start_gather(ci, b):
            s2r, tr, gsem = b
            pltpu.async_copy(s2_hbm.at[dstv.at[ci]], s2r, gsem)
            pltpu.async_copy(t_hbm.at[srcv.at[ci]], tr, gsem)

        def wait_gather(ci, b):
            s2r, tr, gsem = b
            pltpu.make_async_copy(s2_hbm.at[dstv.at[ci]], s2r, gsem).wait()
            pltpu.make_async_copy(t_hbm.at[srcv.at[ci]], tr, gsem).wait()

        def start_scatter(ci):
            pltpu.async_copy(comp, acc.at[dstv.at[ci]], ssem, add=True)

        def wait_scatter(ci):
            pltpu.make_async_copy(comp, acc.at[dstv.at[ci]], ssem).wait()

        def compute(b):
            s2r, tr, _ = b

            @plsc.parallel_loop(0, _CH, 1, unroll=8)
            def edge(e):
                ev = tr[e, pl.ds(wacc, 16)] + s2r[e]
                ev = jnp.maximum(ev, _ALPHA * ev)
                exv = jnp.exp(ev)
                for j, hk in enumerate(chunk_heads):
                    t = tr[e, pl.ds(16 * j, 16)]
                    if hk is None:
                        comp[e, pl.ds(16 * j, 16)] = exv * t
                    else:
                        comp[e, pl.ds(16 * j, 16)] = exv[hk] * t

        def zrow(i, carry):
            for j in range(nck):
                comp[i, pl.ds(16 * j, 16)] = jnp.zeros((16,), jnp.float32)
            return carry
        lax.fori_loop(0, _CH, zrow, 0)

        r0 = s * _RPB
        off = 0
        while off < _RPB:
            n = min(_CH, _RPB - off)
            pltpu.sync_copy(comp.at[pl.ds(0, n)], acc.at[pl.ds(r0 + off, n)])
            off += n

        @pl.when(s == _NSUB - 1)
        def _clear_tail():
            pltpu.sync_copy(comp.at[pl.ds(0, _RTAIL)],
                            acc.at[pl.ds(_NSUB * _RPB, _RTAIL)])

        pltpu.sync_copy(src_hbm.at[pl.ds(w * _NCHUNK, _NCHUNK)], srcv)
        pltpu.sync_copy(dst_hbm.at[pl.ds(w * _NCHUNK, _NCHUNK)], dstv)
        plsc.subcore_barrier()

        start_scatter(0)
        start_gather(0, bufs[0])
        start_gather(1, bufs[1])

        def pipe(i, carry):
            c0 = i * 2
            c1 = c0 + 1
            wait_gather(c0, bufs[0])
            wait_scatter(c0)
            compute(bufs[0])
            start_scatter(c0)

            @pl.when(i < (_NCHUNK // 2) - 1)
            def _pf0():
                start_gather(c0 + 2, bufs[0])

            wait_gather(c1, bufs[1])
            wait_scatter(c1)
            compute(bufs[1])
            start_scatter(c1)

            @pl.when(i < (_NCHUNK // 2) - 1)
            def _pf1():
                start_gather(c1 + 2, bufs[1])
            return carry
        lax.fori_loop(0, _NCHUNK // 2, pipe, 0)

        wait_scatter(_NCHUNK - 1)
        plsc.subcore_barrier()
        off = 0
        while off < _RPB:
            n = min(_CH, _RPB - off)
            pltpu.sync_copy(acc.at[pl.ds(r0 + off, n)],
                            out_hbm.at[c, pl.ds(r0 + off, n)])
            off += n

        @pl.when(s == _NSUB - 1)
        def _out_tail():
            pltpu.sync_copy(acc.at[pl.ds(_NSUB * _RPB, _RTAIL)],
                            out_hbm.at[c, pl.ds(_NSUB * _RPB, _RTAIL)])

    return sc_edge


_sc_edge_l1 = _make_sc_edge(_T1, _A1, [0, 1, 2, 3, 4, 5, 6, 7, None])
_sc_edge_l2 = _make_sc_edge(_T2, _A2, [0, 0, 0])

_BN = 2000


def _tc1_body(x_ref, wpad_ref, b_ref, m2_ref, h_ref, s2_ref):
    xx = x_ref[...]
    h_ref[...] = (jnp.dot(xx, wpad_ref[...],
                          preferred_element_type=jnp.float32) + b_ref[...])
    s2_ref[...] = jnp.dot(xx, m2_ref[...], preferred_element_type=jnp.float32)


def _tc1(x, wpad, b1, m2):
    return pl.pallas_call(
        _tc1_body,
        grid=(_N // _BN,),
        in_specs=[
            pl.BlockSpec((_BN, _F), lambda i: (i, 0)),
            pl.BlockSpec((_F, _T1), lambda i: (0, 0)),
            pl.BlockSpec((1, _T1), lambda i: (0, 0)),
            pl.BlockSpec((_F, 16), lambda i: (0, 0)),
        ],
        out_specs=[
            pl.BlockSpec((_BN, _T1), lambda i: (i, 0)),
            pl.BlockSpec((_BN, 16), lambda i: (i, 0)),
        ],
        out_shape=[
            jax.ShapeDtypeStruct((_N, _T1), jnp.float32),
            jax.ShapeDtypeStruct((_N, 16), jnp.float32),
        ],
    )(x, wpad, b1, m2)


def _tc2_body(p_ref, r8_ref, w2_ref, b2_ref, m2_ref, h2_ref, s2_ref):
    nsum = p_ref[0] + p_ref[1]
    num = nsum[:, :_F]
    den8 = nsum[:, _F:_F + _NH]
    den = jnp.dot(den8, r8_ref[...], preferred_element_type=jnp.float32)
    feat = num / (den + 1e-16)
    feat = jnp.where(feat > 0.0, feat, jnp.exp(feat) - 1.0)
    h2_ref[...] = (jnp.dot(feat, w2_ref[...],
                           preferred_element_type=jnp.float32) + b2_ref[...])
    s2_ref[...] = jnp.dot(feat, m2_ref[...], preferred_element_type=jnp.float32)


def _tc2(p1, r8, w2pad, b2, m2b):
    return pl.pallas_call(
        _tc2_body,
        grid=(_N // _BN,),
        in_specs=[
            pl.BlockSpec((_NCORE, _BN, _A1), lambda i: (0, i, 0)),
            pl.BlockSpec((_NH, _F), lambda i: (0, 0)),
            pl.BlockSpec((_F, _T2), lambda i: (0, 0)),
            pl.BlockSpec((1, _T2), lambda i: (0, 0)),
            pl.BlockSpec((_F, 16), lambda i: (0, 0)),
        ],
        out_specs=[
            pl.BlockSpec((_BN, _T2), lambda i: (i, 0)),
            pl.BlockSpec((_BN, 16), lambda i: (i, 0)),
        ],
        out_shape=[
            jax.ShapeDtypeStruct((_N, _T2), jnp.float32),
            jax.ShapeDtypeStruct((_N, 16), jnp.float32),
        ],
    )(p1, r8, w2pad, b2, m2b)


def _tc3_body(p_ref, o_ref):
    n2 = p_ref[0] + p_ref[1]
    num = n2[:, :_NC]
    den = n2[:, _NC:_NC + 1]
    o = num / (den + 1e-16)
    o = jnp.where(o > 0.0, o, jnp.exp(o) - 1.0)
    pooled = jnp.mean(o, axis=0, keepdims=True)
    m = jnp.max(pooled, axis=1, keepdims=True)
    z = pooled - m
    lse = jnp.log(jnp.sum(jnp.exp(z), axis=1, keepdims=True))
    o_ref[...] = z - lse


def _tc3(p2):
    return pl.pallas_call(
        _tc3_body,
        out_shape=jax.ShapeDtypeStruct((1, _NC), jnp.float32),
    )(p2)


def kernel(x, edge_index, Ws, a_att, W_out, a_out):
    f32 = jnp.float32
    src = edge_index[0].astype(jnp.int32).reshape(_E // _CH, _CH)
    dst = edge_index[1].astype(jnp.int32).reshape(_E // _CH, _CH)

    wcat = jnp.transpose(Ws, (1, 0, 2)).reshape(_F, _NH * _HID)
    a1 = a_att[:, :_HID, 0]
    a2 = a_att[:, _HID:, 0]
    m1 = jnp.einsum("kfj,kj->fk", Ws, a1)
    m2 = jnp.pad(jnp.einsum("kfj,kj->fk", Ws, a2), ((0, 0), (0, 16 - _NH)))
    wpad = jnp.zeros((_F, _T1), f32)
    wpad = wpad.at[:, :_NH * _HID].set(wcat)
    wpad = wpad.at[:, _A1:_A1 + _NH].set(m1)
    b1 = jnp.zeros((1, _T1), f32).at[0, _NH * _HID:_NH * _HID + _NH].set(1.0)
    r8 = jnp.repeat(jnp.eye(_NH, dtype=f32), _HID, axis=1)

    w2pad = jnp.zeros((_F, _T2), f32)
    w2pad = w2pad.at[:, :_NC].set(W_out)
    w2pad = w2pad.at[:, _A2].set(W_out @ a_out[:_NC, 0])
    b2 = jnp.zeros((1, _T2), f32).at[0, _NC].set(1.0)
    m2b = jnp.pad((W_out @ a_out[_NC:, 0])[:, None], ((0, 0), (0, 15)))

    h1t, s2v = _tc1(x, wpad, b1, m2)
    p1 = _sc_edge_l1(src, dst, s2v, h1t)
    h2t, s2b = _tc2(p1, r8, w2pad, b2, m2b)
    p2 = _sc_edge_l2(src, dst, s2b, h2t)
    return _tc3(p2)


# speedup vs baseline: 142.1780x; 1.1376x over previous
import functools

import jax
import jax.numpy as jnp
from jax import lax
from jax.experimental import pallas as pl
from jax.experimental.pallas import tpu as pltpu
from jax.experimental.pallas import tpu_sc as plsc

_N = 10000
_E = 320000
_F = 128
_HID = 16
_NH = 8
_NC = 40
_ALPHA = 0.2

_NCORE = 2
_NSUB = 16
_NW = _NCORE * _NSUB
_EPW = _E // _NW
_RPB = 624
_RTAIL = _N - _NSUB * _RPB

_A1 = 144
_T1 = 144
_E1 = 128
_C1 = 40
_A2 = 48
_T2 = 48
_E2 = 32
_C2 = 100


def _make_sc_edge(wtab, wacc, ev_off, ch, chunk_heads):
    nck = wacc // 16
    assert len(chunk_heads) == nck
    nchunk = _EPW // ch
    assert nchunk * ch == _EPW and nchunk % 2 == 0 and ch <= 128

    mesh = plsc.VectorSubcoreMesh(core_axis_name="c", subcore_axis_name="s")

    @functools.partial(
        pl.kernel,
        out_type=jax.ShapeDtypeStruct((_NCORE, _N, wacc), jnp.float32),
        mesh=mesh,
        compiler_params=pltpu.CompilerParams(use_tc_tiling_on_sc=False),
        scratch_types=[
            pltpu.VMEM((nchunk, ch), jnp.int32),
            pltpu.VMEM((nchunk, ch), jnp.int32),
            pltpu.VMEM((ch, 16), jnp.float32),
            pltpu.VMEM((ch, wtab), jnp.float32),
            pltpu.VMEM((ch, 16), jnp.float32),
            pltpu.VMEM((ch, wtab), jnp.float32),
            pltpu.VMEM((ch, wacc), jnp.float32),
            pltpu.SemaphoreType.DMA,
            pltpu.SemaphoreType.DMA,
            pltpu.SemaphoreType.DMA,
            pltpu.VMEM_SHARED((_N, wacc), jnp.float32),
        ],
    )
    def sc_edge(src_hbm, dst_hbm, s2_hbm, t_hbm, out_hbm,
                srcv, dstv, s2r0, tr0, s2r1, tr1, comp,
                gsem0, gsem1, ssem, acc):
        c = lax.axis_index("c")
        s = lax.axis_index("s")
        w = c * _NSUB + s

        bufs = ((s2r0, tr0, gsem0), (s2r1, tr1, gsem1))

        def start_gather(ci, b):
            s2r, tr, gsem = b
            pltpu.async_copy(s2_hbm.at[dstv.at[ci]], s2r, gsem)
            pltpu.async_copy(t_hbm.at[srcv.at[ci]], tr, gsem)

        def wait_gather(ci, b):
            s2r, tr, gsem = b
            pltpu.make_async_copy(s2_hbm.at[dstv.at[ci]], s2r, gsem).wait()
            pltpu.make_async_copy(t_hbm.at[srcv.at[ci]], tr, gsem).wait()

        def start_scatter(ci):
            pltpu.async_copy(comp, acc.at[dstv.at[ci]], ssem, add=True)

        def wait_scatter(ci):
            pltpu.make_async_copy(comp, acc.at[dstv.at[ci]], ssem).wait()

        def compute(b):
            s2r, tr, _ = b

            @plsc.parallel_loop(0, ch, 1, unroll=8)
            def edge(e):
                ev = tr[e, pl.ds(ev_off, 16)] + s2r[e]
                ev = jnp.maximum(ev, _ALPHA * ev)
                exv = jnp.exp(ev)
                for j, hk in enumerate(chunk_heads):
                    if hk is None:
                        comp[e, pl.ds(16 * j, 16)] = exv
                    else:
                        t = tr[e, pl.ds(16 * j, 16)]
                        comp[e, pl.ds(16 * j, 16)] = exv[hk] * t

        def zrow(i, carry):
            for j in range(nck):
                comp[i, pl.ds(16 * j, 16)] = jnp.zeros((16,), jnp.float32)
            return carry
        lax.fori_loop(0, ch, zrow, 0)

        r0 = s * _RPB
        off = 0
        while off < _RPB:
            n = min(ch, _RPB - off)
            pltpu.sync_copy(comp.at[pl.ds(0, n)], acc.at[pl.ds(r0 + off, n)])
            off += n

        @pl.when(s == _NSUB - 1)
        def _clear_tail():
            pltpu.sync_copy(comp.at[pl.ds(0, _RTAIL)],
                            acc.at[pl.ds(_NSUB * _RPB, _RTAIL)])

        pltpu.sync_copy(src_hbm.at[pl.ds(w * nchunk, nchunk)], srcv)
        pltpu.sync_copy(dst_hbm.at[pl.ds(w * nchunk, nchunk)], dstv)
        plsc.subcore_barrier()

        start_scatter(0)
        start_gather(0, bufs[0])
        start_gather(1, bufs[1])

        def pipe(i, carry):
            c0 = i * 2
            c1 = c0 + 1
            wait_gather(c0, bufs[0])
            wait_scatter(c0)
            compute(bufs[0])
            start_scatter(c0)

            @pl.when(i < (nchunk // 2) - 1)
            def _pf0():
                start_gather(c0 + 2, bufs[0])

            wait_gather(c1, bufs[1])
            wait_scatter(c1)
            compute(bufs[1])
            start_scatter(c1)

            @pl.when(i < (nchunk // 2) - 1)
            def _pf1():
                start_gather(c1 + 2, bufs[1])
            return carry
        lax.fori_loop(0, nchunk // 2, pipe, 0)

        wait_scatter(nchunk - 1)
        plsc.subcore_barrier()
        off = 0
        while off < _RPB:
            n = min(ch, _RPB - off)
            pltpu.sync_copy(acc.at[pl.ds(r0 + off, n)],
                            out_hbm.at[c, pl.ds(r0 + off, n)])
            off += n

        @pl.when(s == _NSUB - 1)
        def _out_tail():
            pltpu.sync_copy(acc.at[pl.ds(_NSUB * _RPB, _RTAIL)],
                            out_hbm.at[c, pl.ds(_NSUB * _RPB, _RTAIL)])

    return sc_edge


_sc_edge_l1 = _make_sc_edge(_T1, _A1, _E1, _C1, [0, 1, 2, 3, 4, 5, 6, 7, None])
_sc_edge_l2 = _make_sc_edge(_T2, _A2, _E2, _C2, [15, 15, 15])

_BN = 2000


def _tc1_body(x_ref, wpad_ref, b_ref, m2_ref, h_ref, s2_ref):
    xx = x_ref[...]
    h_ref[...] = (jnp.dot(xx, wpad_ref[...],
                          preferred_element_type=jnp.float32) + b_ref[...])
    s2_ref[...] = jnp.dot(xx, m2_ref[...], preferred_element_type=jnp.float32)


def _tc1(x, wpad, b1, m2):
    return pl.pallas_call(
        _tc1_body,
        grid=(_N // _BN,),
        in_specs=[
            pl.BlockSpec((_BN, _F), lambda i: (i, 0)),
            pl.BlockSpec((_F, _T1), lambda i: (0, 0)),
            pl.BlockSpec((1, _T1), lambda i: (0, 0)),
            pl.BlockSpec((_F, 16), lambda i: (0, 0)),
        ],
        out_specs=[
            pl.BlockSpec((_BN, _T1), lambda i: (i, 0)),
            pl.BlockSpec((_BN, 16), lambda i: (i, 0)),
        ],
        out_shape=[
            jax.ShapeDtypeStruct((_N, _T1), jnp.float32),
            jax.ShapeDtypeStruct((_N, 16), jnp.float32),
        ],
    )(x, wpad, b1, m2)


def _tc2_body(p_ref, r8_ref, w2_ref, b2_ref, m2_ref, h2_ref, s2_ref):
    nsum = p_ref[0] + p_ref[1]
    num = nsum[:, :_F]
    den8 = nsum[:, _F:_F + _NH]
    den = jnp.dot(den8, r8_ref[...], preferred_element_type=jnp.float32)
    feat = num / (den + 1e-16)
    feat = jnp.where(feat > 0.0, feat, jnp.exp(feat) - 1.0)
    h2_ref[...] = (jnp.dot(feat, w2_ref[...],
                           preferred_element_type=jnp.float32) + b2_ref[...])
    s2_ref[...] = jnp.dot(feat, m2_ref[...], preferred_element_type=jnp.float32)


def _tc2(p1, r8, w2pad, b2, m2b):
    return pl.pallas_call(
        _tc2_body,
        grid=(_N // _BN,),
        in_specs=[
            pl.BlockSpec((_NCORE, _BN, _A1), lambda i: (0, i, 0)),
            pl.BlockSpec((_NH, _F), lambda i: (0, 0)),
            pl.BlockSpec((_F, _T2), lambda i: (0, 0)),
            pl.BlockSpec((1, _T2), lambda i: (0, 0)),
            pl.BlockSpec((_F, 16), lambda i: (0, 0)),
        ],
        out_specs=[
            pl.BlockSpec((_BN, _T2), lambda i: (i, 0)),
            pl.BlockSpec((_BN, 16), lambda i: (i, 0)),
        ],
        out_shape=[
            jax.ShapeDtypeStruct((_N, _T2), jnp.float32),
            jax.ShapeDtypeStruct((_N, 16), jnp.float32),
        ],
    )(p1, r8, w2pad, b2, m2b)


def _tc3_body(p_ref, o_ref):
    n2 = p_ref[0] + p_ref[1]
    num = n2[:, :_NC]
    den = n2[:, _NC:_NC + 1]
    o = num / (den + 1e-16)
    o = jnp.where(o > 0.0, o, jnp.exp(o) - 1.0)
    pooled = jnp.mean(o, axis=0, keepdims=True)
    m = jnp.max(pooled, axis=1, keepdims=True)
    z = pooled - m
    lse = jnp.log(jnp.sum(jnp.exp(z), axis=1, keepdims=True))
    o_ref[...] = z - lse


def _tc3(p2):
    return pl.pallas_call(
        _tc3_body,
        out_shape=jax.ShapeDtypeStruct((1, _NC), jnp.float32),
    )(p2)


def kernel(x, edge_index, Ws, a_att, W_out, a_out):
    f32 = jnp.float32
    e0 = edge_index[0].astype(jnp.int32)
    e1 = edge_index[1].astype(jnp.int32)
    src1 = e0.reshape(_E // _C1, _C1)
    dst1 = e1.reshape(_E // _C1, _C1)
    src2 = e0.reshape(_E // _C2, _C2)
    dst2 = e1.reshape(_E // _C2, _C2)

    wcat = jnp.transpose(Ws, (1, 0, 2)).reshape(_F, _NH * _HID)
    a1 = a_att[:, :_HID, 0]
    a2 = a_att[:, _HID:, 0]
    m1 = jnp.einsum("kfj,kj->fk", Ws, a1)
    m2 = jnp.pad(jnp.einsum("kfj,kj->fk", Ws, a2), ((0, 0), (0, 16 - _NH)))
    wpad = jnp.zeros((_F, _T1), f32)
    wpad = wpad.at[:, :_NH * _HID].set(wcat)
    wpad = wpad.at[:, _E1:_E1 + _NH].set(m1)
    b1 = jnp.zeros((1, _T1), f32).at[0, _E1 + _NH:].set(-1e30)
    r8 = jnp.repeat(jnp.eye(_NH, dtype=f32), _HID, axis=1)

    w2pad = jnp.zeros((_F, _T2), f32)
    w2pad = w2pad.at[:, :_NC].set(W_out)
    w2pad = w2pad.at[:, _T2 - 1].set(W_out @ a_out[:_NC, 0])
    b2 = jnp.zeros((1, _T2), f32).at[0, _NC].set(1.0)
    m2b = jnp.zeros((_F, 16), f32).at[:, 15].set(W_out @ a_out[_NC:, 0])

    h1t, s2v = _tc1(x, wpad, b1, m2)
    p1 = _sc_edge_l1(src1, dst1, s2v, h1t)
    h2t, s2b = _tc2(p1, r8, w2pad, b2, m2b)
    p2 = _sc_edge_l2(src2, dst2, s2b, h2t)
    return _tc3(p2)


# speedup vs baseline: 143.6837x; 1.0106x over previous
import functools

import jax
import jax.numpy as jnp
from jax import lax
from jax.experimental import pallas as pl
from jax.experimental.pallas import tpu as pltpu
from jax.experimental.pallas import tpu_sc as plsc

_N = 10000
_E = 320000
_F = 128
_HID = 16
_NH = 8
_NC = 40
_ALPHA = 0.2

_NCORE = 2
_NSUB = 16
_NW = _NCORE * _NSUB
_EPW = _E // _NW
_RPB = 624
_RTAIL = _N - _NSUB * _RPB

_A1 = 144
_T1 = 144
_E1 = 128
_C1 = 40
_A2 = 48
_T2 = 48
_E2 = 32
_C2 = 100


def _make_sc_edge(wtab, wacc, ev_off, ch, chunk_heads):
    nck = wacc // 16
    assert len(chunk_heads) == nck
    nchunk = _EPW // ch
    assert nchunk * ch == _EPW and nchunk % 2 == 0 and ch <= 128

    mesh = plsc.VectorSubcoreMesh(core_axis_name="c", subcore_axis_name="s")

    @functools.partial(
        pl.kernel,
        out_type=jax.ShapeDtypeStruct((_NCORE, _N, wacc), jnp.float32),
        mesh=mesh,
        compiler_params=pltpu.CompilerParams(use_tc_tiling_on_sc=False),
        scratch_types=[
            pltpu.VMEM((nchunk, ch), jnp.int32),
            pltpu.VMEM((nchunk, ch), jnp.int32),
            pltpu.VMEM((ch, 16), jnp.float32),
            pltpu.VMEM((ch, wtab), jnp.float32),
            pltpu.VMEM((ch, 16), jnp.float32),
            pltpu.VMEM((ch, wtab), jnp.float32),
            pltpu.VMEM((ch, wacc), jnp.float32),
            pltpu.SemaphoreType.DMA,
            pltpu.SemaphoreType.DMA,
            pltpu.SemaphoreType.DMA,
            pltpu.VMEM_SHARED((_N, wacc), jnp.float32),
        ],
    )
    def sc_edge(src_hbm, dst_hbm, s2_hbm, t_hbm, out_hbm,
                srcv, dstv, s2r0, tr0, s2r1, tr1, comp,
                gsem0, gsem1, ssem, acc):
        c = lax.axis_index("c")
        s = lax.axis_index("s")
        w = c * _NSUB + s

        bufs = ((s2r0, tr0, gsem0), (s2r1, tr1, gsem1))

        def start_gather(ci, b):
            s2r, tr, gsem = b
            pltpu.async_copy(s2_hbm.at[dstv.at[ci]], s2r, gsem)
            pltpu.async_copy(t_hbm.at[srcv.at[ci]], tr, gsem)

        def wait_gather(ci, b):
            s2r, tr, gsem = b
            pltpu.make_async_copy(s2_hbm.at[dstv.at[ci]], s2r, gsem).wait()
            pltpu.make_async_copy(t_hbm.at[srcv.at[ci]], tr, gsem).wait()

        def start_scatter(ci):
            pltpu.async_copy(comp, acc.at[dstv.at[ci]], ssem, add=True)

        def wait_scatter(ci):
            pltpu.make_async_copy(comp, acc.at[dstv.at[ci]], ssem).wait()

        def compute(b):
            s2r, tr, _ = b

            @plsc.parallel_loop(0, ch, 1, unroll=10)
            def edge(e):
                ev = tr[e, pl.ds(ev_off, 16)] + s2r[e]
                ev = jnp.maximum(ev, _ALPHA * ev)
                exv = jnp.exp(ev)
                for j, hk in enumerate(chunk_heads):
                    if hk is None:
                        comp[e, pl.ds(16 * j, 16)] = exv
                    else:
                        t = tr[e, pl.ds(16 * j, 16)]
                        comp[e, pl.ds(16 * j, 16)] = exv[hk] * t

        def zrow(i, carry):
            for j in range(nck):
                comp[i, pl.ds(16 * j, 16)] = jnp.zeros((16,), jnp.float32)
            return carry
        lax.fori_loop(0, ch, zrow, 0)

        r0 = s * _RPB
        off = 0
        while off < _RPB:
            n = min(ch, _RPB - off)
            pltpu.sync_copy(comp.at[pl.ds(0, n)], acc.at[pl.ds(r0 + off, n)])
            off += n

        @pl.when(s == _NSUB - 1)
        def _clear_tail():
            pltpu.sync_copy(comp.at[pl.ds(0, _RTAIL)],
                            acc.at[pl.ds(_NSUB * _RPB, _RTAIL)])

        pltpu.sync_copy(src_hbm.at[pl.ds(w * nchunk, nchunk)], srcv)
        pltpu.sync_copy(dst_hbm.at[pl.ds(w * nchunk, nchunk)], dstv)
        plsc.subcore_barrier()

        start_scatter(0)
        start_gather(0, bufs[0])
        start_gather(1, bufs[1])

        def pipe(i, carry):
            c0 = i * 2
            c1 = c0 + 1
            wait_gather(c0, bufs[0])
            wait_scatter(c0)
            compute(bufs[0])
            start_scatter(c0)

            @pl.when(i < (nchunk // 2) - 1)
            def _pf0():
                start_gather(c0 + 2, bufs[0])

            wait_gather(c1, bufs[1])
            wait_scatter(c1)
            compute(bufs[1])
            start_scatter(c1)

            @pl.when(i < (nchunk // 2) - 1)
            def _pf1():
                start_gather(c1 + 2, bufs[1])
            return carry
        lax.fori_loop(0, nchunk // 2, pipe, 0)

        wait_scatter(nchunk - 1)
        plsc.subcore_barrier()
        off = 0
        while off < _RPB:
            n = min(ch, _RPB - off)
            pltpu.sync_copy(acc.at[pl.ds(r0 + off, n)],
                            out_hbm.at[c, pl.ds(r0 + off, n)])
            off += n

        @pl.when(s == _NSUB - 1)
        def _out_tail():
            pltpu.sync_copy(acc.at[pl.ds(_NSUB * _RPB, _RTAIL)],
                            out_hbm.at[c, pl.ds(_NSUB * _RPB, _RTAIL)])

    return sc_edge


_sc_edge_l1 = _make_sc_edge(_T1, _A1, _E1, _C1, [0, 1, 2, 3, 4, 5, 6, 7, None])
_sc_edge_l2 = _make_sc_edge(_T2, _A2, _E2, _C2, [15, 15, 15])

_BN = 2000


def _tc1_body(x_ref, wpad_ref, b_ref, m2_ref, h_ref, s2_ref):
    xx = x_ref[...]
    h_ref[...] = (jnp.dot(xx, wpad_ref[...],
                          preferred_element_type=jnp.float32) + b_ref[...])
    s2_ref[...] = jnp.dot(xx, m2_ref[...], preferred_element_type=jnp.float32)


def _tc1(x, wpad, b1, m2):
    return pl.pallas_call(
        _tc1_body,
        grid=(_N // _BN,),
        in_specs=[
            pl.BlockSpec((_BN, _F), lambda i: (i, 0)),
            pl.BlockSpec((_F, _T1), lambda i: (0, 0)),
            pl.BlockSpec((1, _T1), lambda i: (0, 0)),
            pl.BlockSpec((_F, 16), lambda i: (0, 0)),
        ],
        out_specs=[
            pl.BlockSpec((_BN, _T1), lambda i: (i, 0)),
            pl.BlockSpec((_BN, 16), lambda i: (i, 0)),
        ],
        out_shape=[
            jax.ShapeDtypeStruct((_N, _T1), jnp.float32),
            jax.ShapeDtypeStruct((_N, 16), jnp.float32),
        ],
    )(x, wpad, b1, m2)


def _tc2_body(p_ref, r8_ref, w2_ref, b2_ref, m2_ref, h2_ref, s2_ref):
    nsum = p_ref[0] + p_ref[1]
    num = nsum[:, :_F]
    den8 = nsum[:, _F:_F + _NH]
    den = jnp.dot(den8, r8_ref[...], preferred_element_type=jnp.float32)
    feat = num / (den + 1e-16)
    feat = jnp.where(feat > 0.0, feat, jnp.exp(feat) - 1.0)
    h2_ref[...] = (jnp.dot(feat, w2_ref[...],
                           preferred_element_type=jnp.float32) + b2_ref[...])
    s2_ref[...] = jnp.dot(feat, m2_ref[...], preferred_element_type=jnp.float32)


def _tc2(p1, r8, w2pad, b2, m2b):
    return pl.pallas_call(
        _tc2_body,
        grid=(_N // _BN,),
        in_specs=[
            pl.BlockSpec((_NCORE, _BN, _A1), lambda i: (0, i, 0)),
            pl.BlockSpec((_NH, _F), lambda i: (0, 0)),
            pl.BlockSpec((_F, _T2), lambda i: (0, 0)),
            pl.BlockSpec((1, _T2), lambda i: (0, 0)),
            pl.BlockSpec((_F, 16), lambda i: (0, 0)),
        ],
        out_specs=[
            pl.BlockSpec((_BN, _T2), lambda i: (i, 0)),
            pl.BlockSpec((_BN, 16), lambda i: (i, 0)),
        ],
        out_shape=[
            jax.ShapeDtypeStruct((_N, _T2), jnp.float32),
            jax.ShapeDtypeStruct((_N, 16), jnp.float32),
        ],
    )(p1, r8, w2pad, b2, m2b)


def _tc3_body(p_ref, o_ref):
    n2 = p_ref[0] + p_ref[1]
    num = n2[:, :_NC]
    den = n2[:, _NC:_NC + 1]
    o = num / (den + 1e-16)
    o = jnp.where(o > 0.0, o, jnp.exp(o) - 1.0)
    pooled = jnp.mean(o, axis=0, keepdims=True)
    m = jnp.max(pooled, axis=1, keepdims=True)
    z = pooled - m
    lse = jnp.log(jnp.sum(jnp.exp(z), axis=1, keepdims=True))
    o_ref[...] = z - lse


def _tc3(p2):
    return pl.pallas_call(
        _tc3_body,
        out_shape=jax.ShapeDtypeStruct((1, _NC), jnp.float32),
    )(p2)


def kernel(x, edge_index, Ws, a_att, W_out, a_out):
    f32 = jnp.float32
    e0 = edge_index[0].astype(jnp.int32)
    e1 = edge_index[1].astype(jnp.int32)
    src1 = e0.reshape(_E // _C1, _C1)
    dst1 = e1.reshape(_E // _C1, _C1)
    src2 = e0.reshape(_E // _C2, _C2)
    dst2 = e1.reshape(_E // _C2, _C2)

    wcat = jnp.transpose(Ws, (1, 0, 2)).reshape(_F, _NH * _HID)
    a1 = a_att[:, :_HID, 0]
    a2 = a_att[:, _HID:, 0]
    m1 = jnp.einsum("kfj,kj->fk", Ws, a1)
    m2 = jnp.pad(jnp.einsum("kfj,kj->fk", Ws, a2), ((0, 0), (0, 16 - _NH)))
    wpad = jnp.zeros((_F, _T1), f32)
    wpad = wpad.at[:, :_NH * _HID].set(wcat)
    wpad = wpad.at[:, _E1:_E1 + _NH].set(m1)
    b1 = jnp.zeros((1, _T1), f32).at[0, _E1 + _NH:].set(-1e30)
    r8 = jnp.repeat(jnp.eye(_NH, dtype=f32), _HID, axis=1)

    w2pad = jnp.zeros((_F, _T2), f32)
    w2pad = w2pad.at[:, :_NC].set(W_out)
    w2pad = w2pad.at[:, _T2 - 1].set(W_out @ a_out[:_NC, 0])
    b2 = jnp.zeros((1, _T2), f32).at[0, _NC].set(1.0)
    m2b = jnp.zeros((_F, 16), f32).at[:, 15].set(W_out @ a_out[_NC:, 0])

    h1t, s2v = _tc1(x, wpad, b1, m2)
    p1 = _sc_edge_l1(src1, dst1, s2v, h1t)
    h2t, s2b = _tc2(p1, r8, w2pad, b2, m2b)
    p2 = _sc_edge_l2(src2, dst2, s2b, h2t)
    return _tc3(p2)
